# baseline (speedup 1.0000x reference)
import functools

import jax
import jax.numpy as jnp
from jax import lax
from jax.experimental import pallas as pl
from jax.experimental.pallas import tpu as pltpu
from jax.experimental.pallas import tpu_sc as plsc

ALPHA = 0.01
EPS = 1e-5

NC = 2
NS = 16
NW = NC * NS
LANES = 16
CHUNK = 128


def _cdiv(a, b):
  return (a + b - 1) // b


def _deg_kernel(n_nodes, e_pad):
  epw = e_pad // NW
  n_chunks = epw // CHUNK
  rows_per_tile = n_nodes // NS

  @functools.partial(
      pl.kernel,
      out_type=jax.ShapeDtypeStruct((NC, n_nodes, LANES), jnp.float32),
      mesh=plsc.VectorSubcoreMesh(core_axis_name="c", subcore_axis_name="s", num_cores=NC, num_subcores=NS),
      compiler_params=pltpu.CompilerParams(use_tc_tiling_on_sc=False, needs_layout_passes=False),
      scratch_types=[
          pltpu.MemorySpace.VMEM_SHARED((n_nodes, LANES), jnp.float32),
          pltpu.VMEM((rows_per_tile, LANES), jnp.float32),
          pltpu.VMEM((CHUNK, LANES), jnp.float32),
          pltpu.VMEM((CHUNK,), jnp.int32),
          pltpu.VMEM((CHUNK,), jnp.int32),
          pltpu.VMEM((CHUNK,), jnp.float32),
      ],
  )
  def deg(src_hbm, dst_hbm, attr_hbm, part_hbm,
          acc_sh, zero_v, rows_v, src_v, dst_v, attr_v):
    c = lax.axis_index("c")
    s = lax.axis_index("s")
    wid = s * NC + c
    base = wid * epw
    r0 = s * rows_per_tile

    @pl.loop(0, rows_per_tile)
    def _(r):
      zero_v[r, pl.ds(0, LANES)] = jnp.zeros((LANES,), jnp.float32)

    pltpu.sync_copy(zero_v, acc_sh.at[pl.ds(r0, rows_per_tile)])
    plsc.subcore_barrier()

    @pl.loop(0, n_chunks)
    def _(k):
      e0 = base + k * CHUNK
      pltpu.sync_copy(src_hbm.at[pl.ds(e0, CHUNK)], src_v)
      pltpu.sync_copy(dst_hbm.at[pl.ds(e0, CHUNK)], dst_v)
      pltpu.sync_copy(attr_hbm.at[pl.ds(e0, CHUNK)], attr_v)

      @pl.loop(0, CHUNK // LANES)
      def _(g):
        sl = pl.ds(g * LANES, LANES)
        ew = jnp.where(src_v[sl] == dst_v[sl], 0.0, attr_v[sl])
        for l in range(LANES):
          rows_v[g * LANES + l, pl.ds(0, LANES)] = jnp.full(
              (LANES,), ew[l], jnp.float32)

      pltpu.sync_copy(rows_v, acc_sh.at[src_v], add=True)

    plsc.subcore_barrier()
    pltpu.sync_copy(acc_sh.at[pl.ds(r0, rows_per_tile)],
                    part_hbm.at[c, pl.ds(r0, rows_per_tile)])

  return deg


def _norm_kernel(n_nodes, e_pad):
  epw = e_pad // NW
  n_chunks = epw // CHUNK

  @functools.partial(
      pl.kernel,
      out_type=jax.ShapeDtypeStruct((e_pad,), jnp.float32),
      mesh=plsc.VectorSubcoreMesh(core_axis_name="c", subcore_axis_name="s", num_cores=NC, num_subcores=NS),
      compiler_params=pltpu.CompilerParams(use_tc_tiling_on_sc=False, needs_layout_passes=False),
      scratch_types=[
          pltpu.VMEM((n_nodes,), jnp.float32),
          pltpu.VMEM((CHUNK,), jnp.int32),
          pltpu.VMEM((CHUNK,), jnp.int32),
          pltpu.VMEM((CHUNK,), jnp.float32),
          pltpu.VMEM((CHUNK,), jnp.float32),
      ],
  )
  def norm(src_hbm, dst_hbm, attr_hbm, dis_hbm, norm_hbm,
           dis_v, src_v, dst_v, attr_v, norm_v):
    c = lax.axis_index("c")
    s = lax.axis_index("s")
    wid = s * NC + c
    base = wid * epw
    pltpu.sync_copy(dis_hbm, dis_v)

    @pl.loop(0, n_chunks)
    def _(k):
      e0 = base + k * CHUNK
      pltpu.sync_copy(src_hbm.at[pl.ds(e0, CHUNK)], src_v)
      pltpu.sync_copy(dst_hbm.at[pl.ds(e0, CHUNK)], dst_v)
      pltpu.sync_copy(attr_hbm.at[pl.ds(e0, CHUNK)], attr_v)
      for j in range(CHUNK // LANES):
        sl = pl.ds(j * LANES, LANES)
        si = src_v[sl]
        di = dst_v[sl]
        av = attr_v[sl]
        dsrc = plsc.load_gather(dis_v, [si])
        ddst = plsc.load_gather(dis_v, [di])
        ew = jnp.where(si == di, 0.0, av)
        norm_v[sl] = -dsrc * ew * ddst
      pltpu.sync_copy(norm_v, norm_hbm.at[pl.ds(e0, CHUNK)])

  return norm


def _lap_kernel(n_nodes, e_pad, dh):
  epw = e_pad // NS
  n_chunks = epw // CHUNK
  rows_per_tile = n_nodes // NS
  zrows = 125

  @functools.partial(
      pl.kernel,
      out_type=jax.ShapeDtypeStruct((n_nodes, NC, dh), jnp.float32),
      mesh=plsc.VectorSubcoreMesh(core_axis_name="c", subcore_axis_name="s", num_cores=NC, num_subcores=NS),
      compiler_params=pltpu.CompilerParams(use_tc_tiling_on_sc=False, needs_layout_passes=False),
      scratch_types=[
          pltpu.MemorySpace.VMEM_SHARED((n_nodes, dh), jnp.float32),
          pltpu.VMEM((CHUNK, dh), jnp.float32),
          pltpu.VMEM((CHUNK, dh), jnp.float32),
          pltpu.VMEM((n_chunks, CHUNK), jnp.int32),
          pltpu.VMEM((2, CHUNK), jnp.int32),
          pltpu.VMEM((2, CHUNK), jnp.float32),
          pltpu.SemaphoreType.DMA,
          pltpu.SemaphoreType.DMA,
          pltpu.SemaphoreType.DMA,
          pltpu.SemaphoreType.DMA,
          pltpu.SemaphoreType.DMA,
          pltpu.SemaphoreType.DMA,
      ],
  )
  def lap(v2_hbm, gidx_hbm, dst_hbm, norm_hbm, out_hbm,
          acc_sh, rows0, rows1, idx_a, dst_b, norm_b,
          gsem0, gsem1, ssem0, ssem1, nsem0, nsem1):
    c = lax.axis_index("c")
    s = lax.axis_index("s")
    r0 = s * rows_per_tile

    pltpu.sync_copy(gidx_hbm.at[c, s], idx_a)

    @pl.loop(0, zrows)
    def _(r):
      for j in range(dh // LANES):
        rows0[r, pl.ds(j * LANES, LANES)] = jnp.zeros((LANES,), jnp.float32)

    for z in range(rows_per_tile // zrows):
      pltpu.sync_copy(rows0.at[pl.ds(0, zrows)],
                      acc_sh.at[pl.ds(r0 + z * zrows, zrows)])
    plsc.subcore_barrier()

    def gather(k, rows, sem):
      pltpu.async_copy(v2_hbm.at[idx_a.at[k]], rows, sem)

    def gwait(k, rows, sem):
      pltpu.make_async_copy(v2_hbm.at[idx_a.at[k]], rows, sem).wait()

    def ndfetch(k, b, sem):
      pltpu.async_copy(dst_hbm.at[s, k], dst_b.at[b], sem)
      pltpu.async_copy(norm_hbm.at[s, k], norm_b.at[b], sem)

    def ndwait(k, b, sem):
      pltpu.make_async_copy(dst_hbm.at[s, k], dst_b.at[b], sem).wait()
      pltpu.make_async_copy(norm_hbm.at[s, k], norm_b.at[b], sem).wait()

    def scat(k, b, rows, sem):
      pltpu.async_copy(rows, acc_sh.at[dst_b.at[b]], sem, add=True)

    def swait(k, b, rows, sem):
      pltpu.make_async_copy(rows, acc_sh.at[dst_b.at[b]], sem).wait()

    def mul(b, rows):
      @plsc.parallel_loop(0, CHUNK // LANES, unroll=2)
      def _(g):
        nv = norm_b[b, pl.ds(g * LANES, LANES)]
        for l in range(LANES):
          w = nv[l]
          r = g * LANES + l
          for j in range(dh // LANES):
            sl = pl.ds(j * LANES, LANES)
            rows[r, sl] = rows[r, sl] * w

    ndfetch(0, 0, nsem0)
    ndfetch(1, 1, nsem1)
    gather(0, rows0, gsem0)
    gather(1, rows1, gsem1)

    @pl.loop(0, n_chunks - 2, step=2)
    def _(k0):
      gwait(k0, rows0, gsem0)
      ndwait(k0, 0, nsem0)
      mul(0, rows0)
      scat(k0, 0, rows0, ssem0)
      gwait(k0 + 1, rows1, gsem1)
      ndwait(k0 + 1, 1, nsem1)
      mul(1, rows1)
      scat(k0 + 1, 1, rows1, ssem1)
      swait(k0, 0, rows0, ssem0)
      ndfetch(k0 + 2, 0, nsem0)
      gather(k0 + 2, rows0, gsem0)
      swait(k0 + 1, 1, rows1, ssem1)
      ndfetch(k0 + 3, 1, nsem1)
      gather(k0 + 3, rows1, gsem1)

    kl = n_chunks - 2
    gwait(kl, rows0, gsem0)
    ndwait(kl, 0, nsem0)
    mul(0, rows0)
    scat(kl, 0, rows0, ssem0)
    gwait(kl + 1, rows1, gsem1)
    ndwait(kl + 1, 1, nsem1)
    mul(1, rows1)
    scat(kl + 1, 1, rows1, ssem1)
    swait(kl, 0, rows0, ssem0)
    swait(kl + 1, 1, rows1, ssem1)

    plsc.subcore_barrier()
    for z in range(rows_per_tile // zrows):
      rr = r0 + z * zrows
      pltpu.sync_copy(acc_sh.at[pl.ds(rr, zrows)],
                      out_hbm.at[pl.ds(rr, zrows), c])

  return lap


def _dis_body(part_ref, dis_ref):
  deg = part_ref[0, :, 0] + part_ref[1, :, 0]
  deg = deg[None, :]
  safe = jax.lax.rsqrt(jnp.maximum(deg, 1e-12))
  dis_ref[...] = jnp.where(deg > 0, safe, 0.0)


def _mm_init_body(x_ref, l_ref, w0_ref, w1_ref, o_ref):
  o_ref[...] = (
      jnp.dot(x_ref[...], w0_ref[...], preferred_element_type=jnp.float32)
      + jnp.dot(l_ref[...], w1_ref[...], preferred_element_type=jnp.float32))


def _mm_step_body(tprev_ref, l_ref, w_ref, acc_ref, t_ref, o_ref):
  t = 2.0 * l_ref[...] - tprev_ref[...]
  t_ref[...] = t
  o_ref[...] = acc_ref[...] + jnp.dot(
      t, w_ref[...], preferred_element_type=jnp.float32)


def _bn_body(acc_ref, b_ref, g_ref, bt_ref, o_ref):
  n = acc_ref.shape[0]
  t = acc_ref[...] + b_ref[...]
  mean = jnp.sum(t, axis=0, keepdims=True) / n
  d = t - mean
  var = jnp.sum(d * d, axis=0, keepdims=True) / n
  xn = d * jax.lax.rsqrt(var + EPS) * g_ref[...] + bt_ref[...]
  o_ref[...] = jnp.where(xn > 0, xn, ALPHA * xn)


def kernel(x, edge_idx, edge_attr, W, b, gamma, beta):
  n, din = x.shape
  kk, _, dout = W.shape
  e = edge_idx.shape[1]
  dh = din // NC

  e_pad = _cdiv(e, NW * CHUNK) * NW * CHUNK
  pad = e_pad - e
  src = jnp.concatenate([edge_idx[0], jnp.zeros((pad,), jnp.int32)])
  dst = jnp.concatenate([edge_idx[1], jnp.zeros((pad,), jnp.int32)])
  attr = jnp.concatenate([edge_attr, jnp.zeros((pad,), jnp.float32)])
  gidx = jnp.stack([NC * src, NC * src + 1])

  part = _deg_kernel(n, e_pad)(src, dst, attr)

  dis = pl.pallas_call(
      _dis_body,
      out_shape=jax.ShapeDtypeStruct((1, n), jnp.float32),
  )(part).reshape((n,))

  norm = _norm_kernel(n, e_pad)(src, dst, attr, dis)

  lap = _lap_kernel(n, e_pad, dh)
  n_chunks = e_pad // NS // CHUNK
  gidx_l = gidx.reshape(NC, NS, n_chunks, CHUNK)
  dst_l = dst.reshape(NS, n_chunks, CHUNK)
  norm_l = norm.reshape(NS, n_chunks, CHUNK)

  def lap_apply(v):
    return lap(v.reshape((NC * n, dh)), gidx_l, dst_l, norm_l).reshape((n, din))

  blk = 2000
  grid = n // blk
  row_spec = pl.BlockSpec((blk, din), lambda i: (i, 0))
  w_spec = pl.BlockSpec((din, dout), lambda i: (0, 0))

  t1 = lap_apply(x)
  out = pl.pallas_call(
      _mm_init_body,
      grid=(grid,),
      in_specs=[row_spec, row_spec, w_spec, w_spec],
      out_specs=row_spec,
      out_shape=jax.ShapeDtypeStruct((n, dout), jnp.float32),
  )(x, t1, W[0], W[1])

  tprev, tcur = x, t1
  for k in range(2, kk):
    lk = lap_apply(tcur)
    tnext, out = pl.pallas_call(
        _mm_step_body,
        grid=(grid,),
        in_specs=[row_spec, row_spec, w_spec, row_spec],
        out_specs=[row_spec, row_spec],
        out_shape=[
            jax.ShapeDtypeStruct((n, din), jnp.float32),
            jax.ShapeDtypeStruct((n, dout), jnp.float32),
        ],
    )(tprev, lk, W[k], out)
    tprev, tcur = tcur, tnext

  vec_spec = pl.BlockSpec((1, dout), lambda: (0, 0))
  full_spec = pl.BlockSpec((n, dout), lambda: (0, 0))
  return pl.pallas_call(
      _bn_body,
      in_specs=[full_spec, vec_spec, vec_spec, vec_spec],
      out_specs=full_spec,
      out_shape=jax.ShapeDtypeStruct((n, dout), jnp.float32),
  )(out, b.reshape(1, dout), gamma.reshape(1, dout), beta.reshape(1, dout))


# speedup vs baseline: 1.0128x; 1.0128x over previous
import functools

import jax
import jax.numpy as jnp
from jax import lax
from jax.experimental import pallas as pl
from jax.experimental.pallas import tpu as pltpu
from jax.experimental.pallas import tpu_sc as plsc

ALPHA = 0.01
EPS = 1e-5

NC = 2
NS = 16
NW = NC * NS
LANES = 16
CHUNK = 128
LAP_CHUNK = 80


def _cdiv(a, b):
  return (a + b - 1) // b


def _lcm(a, b):
  import math
  return a * b // math.gcd(a, b)


def _deg_kernel(n_nodes, e_pad):
  epw = e_pad // NW
  n_chunks = epw // CHUNK
  rows_per_tile = n_nodes // NS

  @functools.partial(
      pl.kernel,
      out_type=jax.ShapeDtypeStruct((NC, n_nodes, LANES), jnp.float32),
      mesh=plsc.VectorSubcoreMesh(core_axis_name="c", subcore_axis_name="s", num_cores=NC, num_subcores=NS),
      compiler_params=pltpu.CompilerParams(use_tc_tiling_on_sc=False, needs_layout_passes=False),
      scratch_types=[
          pltpu.MemorySpace.VMEM_SHARED((n_nodes, LANES), jnp.float32),
          pltpu.VMEM((rows_per_tile, LANES), jnp.float32),
          pltpu.VMEM((CHUNK, LANES), jnp.float32),
          pltpu.VMEM((CHUNK,), jnp.int32),
          pltpu.VMEM((CHUNK,), jnp.int32),
          pltpu.VMEM((CHUNK,), jnp.float32),
      ],
  )
  def deg(src_hbm, dst_hbm, attr_hbm, part_hbm,
          acc_sh, zero_v, rows_v, src_v, dst_v, attr_v):
    c = lax.axis_index("c")
    s = lax.axis_index("s")
    wid = s * NC + c
    base = wid * epw
    r0 = s * rows_per_tile

    @pl.loop(0, rows_per_tile)
    def _(r):
      zero_v[r, pl.ds(0, LANES)] = jnp.zeros((LANES,), jnp.float32)

    pltpu.sync_copy(zero_v, acc_sh.at[pl.ds(r0, rows_per_tile)])
    plsc.subcore_barrier()

    @pl.loop(0, n_chunks)
    def _(k):
      e0 = base + k * CHUNK
      pltpu.sync_copy(src_hbm.at[pl.ds(e0, CHUNK)], src_v)
      pltpu.sync_copy(dst_hbm.at[pl.ds(e0, CHUNK)], dst_v)
      pltpu.sync_copy(attr_hbm.at[pl.ds(e0, CHUNK)], attr_v)

      @pl.loop(0, CHUNK // LANES)
      def _(g):
        sl = pl.ds(g * LANES, LANES)
        ew = jnp.where(src_v[sl] == dst_v[sl], 0.0, attr_v[sl])
        for l in range(LANES):
          rows_v[g * LANES + l, pl.ds(0, LANES)] = jnp.full(
              (LANES,), ew[l], jnp.float32)

      pltpu.sync_copy(rows_v, acc_sh.at[src_v], add=True)

    plsc.subcore_barrier()
    pltpu.sync_copy(acc_sh.at[pl.ds(r0, rows_per_tile)],
                    part_hbm.at[c, pl.ds(r0, rows_per_tile)])

  return deg


def _norm_kernel(n_nodes, e_pad):
  epw = e_pad // NW
  n_chunks = epw // CHUNK

  @functools.partial(
      pl.kernel,
      out_type=jax.ShapeDtypeStruct((e_pad,), jnp.float32),
      mesh=plsc.VectorSubcoreMesh(core_axis_name="c", subcore_axis_name="s", num_cores=NC, num_subcores=NS),
      compiler_params=pltpu.CompilerParams(use_tc_tiling_on_sc=False, needs_layout_passes=False),
      scratch_types=[
          pltpu.VMEM((n_nodes,), jnp.float32),
          pltpu.VMEM((CHUNK,), jnp.int32),
          pltpu.VMEM((CHUNK,), jnp.int32),
          pltpu.VMEM((CHUNK,), jnp.float32),
          pltpu.VMEM((CHUNK,), jnp.float32),
      ],
  )
  def norm(src_hbm, dst_hbm, attr_hbm, dis_hbm, norm_hbm,
           dis_v, src_v, dst_v, attr_v, norm_v):
    c = lax.axis_index("c")
    s = lax.axis_index("s")
    wid = s * NC + c
    base = wid * epw
    pltpu.sync_copy(dis_hbm, dis_v)

    @pl.loop(0, n_chunks)
    def _(k):
      e0 = base + k * CHUNK
      pltpu.sync_copy(src_hbm.at[pl.ds(e0, CHUNK)], src_v)
      pltpu.sync_copy(dst_hbm.at[pl.ds(e0, CHUNK)], dst_v)
      pltpu.sync_copy(attr_hbm.at[pl.ds(e0, CHUNK)], attr_v)
      for j in range(CHUNK // LANES):
        sl = pl.ds(j * LANES, LANES)
        si = src_v[sl]
        di = dst_v[sl]
        av = attr_v[sl]
        dsrc = plsc.load_gather(dis_v, [si])
        ddst = plsc.load_gather(dis_v, [di])
        ew = jnp.where(si == di, 0.0, av)
        norm_v[sl] = -dsrc * ew * ddst
      pltpu.sync_copy(norm_v, norm_hbm.at[pl.ds(e0, CHUNK)])

  return norm


def _lap_kernel(n_nodes, e_pad, dh):
  ck = LAP_CHUNK
  epw = e_pad // NS
  n_chunks = epw // ck
  rows_per_tile = n_nodes // NS
  zrows = 125
  NBUF = 4

  @functools.partial(
      pl.kernel,
      out_type=jax.ShapeDtypeStruct((n_nodes, NC, dh), jnp.float32),
      mesh=plsc.VectorSubcoreMesh(core_axis_name="c", subcore_axis_name="s", num_cores=NC, num_subcores=NS),
      compiler_params=pltpu.CompilerParams(use_tc_tiling_on_sc=False, needs_layout_passes=False),
      scratch_types=[
          pltpu.MemorySpace.VMEM_SHARED((n_nodes, dh), jnp.float32),
          pltpu.VMEM((NBUF, ck, dh), jnp.float32),
          pltpu.VMEM((NBUF, ck), jnp.int32),
          pltpu.VMEM((NBUF, ck), jnp.int32),
          pltpu.VMEM((NBUF, ck), jnp.float32),
          pltpu.SemaphoreType.DMA((NBUF,)),
          pltpu.SemaphoreType.DMA((NBUF,)),
          pltpu.SemaphoreType.DMA((NBUF,)),
      ],
  )
  def lap(v2_hbm, gidx_hbm, dst_hbm, norm_hbm, out_hbm,
          acc_sh, rows, idx_b, dst_b, norm_b, gsem, ssem, nsem):
    c = lax.axis_index("c")
    s = lax.axis_index("s")
    r0 = s * rows_per_tile

    @pl.loop(0, ck)
    def _(r):
      for j in range(dh // LANES):
        rows[0, r, pl.ds(j * LANES, LANES)] = jnp.zeros((LANES,), jnp.float32)

    nfull = rows_per_tile // ck
    rem = rows_per_tile - nfull * ck
    for z in range(nfull):
      pltpu.sync_copy(rows.at[0], acc_sh.at[pl.ds(r0 + z * ck, ck)])
    if rem:
      pltpu.sync_copy(rows.at[0, pl.ds(0, rem)],
                      acc_sh.at[pl.ds(r0 + nfull * ck, rem)])
    plsc.subcore_barrier()

    def fetch(k, b):
      pltpu.async_copy(gidx_hbm.at[c, s, k], idx_b.at[b], nsem.at[b])
      pltpu.async_copy(dst_hbm.at[s, k], dst_b.at[b], nsem.at[b])
      pltpu.async_copy(norm_hbm.at[s, k], norm_b.at[b], nsem.at[b])

    def fwait(k, b):
      pltpu.make_async_copy(gidx_hbm.at[c, s, k], idx_b.at[b], nsem.at[b]).wait()
      pltpu.make_async_copy(dst_hbm.at[s, k], dst_b.at[b], nsem.at[b]).wait()
      pltpu.make_async_copy(norm_hbm.at[s, k], norm_b.at[b], nsem.at[b]).wait()

    def gather(b):
      pltpu.async_copy(v2_hbm.at[idx_b.at[b]], rows.at[b], gsem.at[b])

    def gwait(b):
      pltpu.make_async_copy(v2_hbm.at[idx_b.at[b]], rows.at[b],
                            gsem.at[b]).wait()

    def scat(b):
      pltpu.async_copy(rows.at[b], acc_sh.at[dst_b.at[b]], ssem.at[b],
                       add=True)

    def swait(b):
      pltpu.make_async_copy(rows.at[b], acc_sh.at[dst_b.at[b]],
                            ssem.at[b]).wait()

    def mul(b):
      @plsc.parallel_loop(0, ck // LANES, unroll=1)
      def _(g):
        nv = norm_b[b, pl.ds(g * LANES, LANES)]
        for l in range(LANES):
          w = nv[l]
          r = g * LANES + l
          for j in range(dh // LANES):
            sl = pl.ds(j * LANES, LANES)
            rows[b, r, sl] = rows[b, r, sl] * w

    def process(k, b):
      gwait(b)
      fwait(k, b)
      mul(b)
      scat(b)

    fetch(0, 0)
    fetch(1, 1)
    pltpu.make_async_copy(gidx_hbm.at[c, s, 0], idx_b.at[0], nsem.at[0]).wait()
    gather(0)
    pltpu.make_async_copy(gidx_hbm.at[c, s, 1], idx_b.at[1], nsem.at[1]).wait()
    gather(1)

    def fwait_idx_only(k, b):
      pltpu.make_async_copy(gidx_hbm.at[c, s, k], idx_b.at[b], nsem.at[b]).wait()

    def fwait_nd(k, b):
      pltpu.make_async_copy(dst_hbm.at[s, k], dst_b.at[b], nsem.at[b]).wait()
      pltpu.make_async_copy(norm_hbm.at[s, k], norm_b.at[b], nsem.at[b]).wait()

    def process_nd(k, b):
      gwait(b)
      fwait_nd(k, b)
      mul(b)
      scat(b)

    def refill(k, b):
      swait(b)
      fetch(k, b)
      fwait_idx_only(k, b)
      gather(b)

    process_nd(0, 0)
    fetch(2, 2)
    fwait_idx_only(2, 2)
    gather(2)
    process_nd(1, 1)
    fetch(3, 3)
    fwait_idx_only(3, 3)
    gather(3)

    @pl.loop(2, n_chunks - 2, step=NBUF)
    def _(jj):
      for bb in range(NBUF):
        b = (2 + bb) % NBUF
        j = jj + bb
        process_nd(j, b)
        refill(j + 2, (b + 2) % NBUF)

    process_nd(n_chunks - 2, (n_chunks - 2) % NBUF)
    process_nd(n_chunks - 1, (n_chunks - 1) % NBUF)
    for b in range(NBUF):
      swait(b)

    plsc.subcore_barrier()
    for z in range(rows_per_tile // zrows):
      rr = r0 + z * zrows
      pltpu.sync_copy(acc_sh.at[pl.ds(rr, zrows)],
                      out_hbm.at[pl.ds(rr, zrows), c])

  return lap


def _dis_body(part_ref, dis_ref):
  deg = part_ref[0, :, 0] + part_ref[1, :, 0]
  deg = deg[None, :]
  safe = jax.lax.rsqrt(jnp.maximum(deg, 1e-12))
  dis_ref[...] = jnp.where(deg > 0, safe, 0.0)


def _mm_init_body(x_ref, l_ref, w0_ref, w1_ref, o_ref):
  o_ref[...] = (
      jnp.dot(x_ref[...], w0_ref[...], preferred_element_type=jnp.float32)
      + jnp.dot(l_ref[...], w1_ref[...], preferred_element_type=jnp.float32))


def _mm_step_body(tprev_ref, l_ref, w_ref, acc_ref, t_ref, o_ref):
  t = 2.0 * l_ref[...] - tprev_ref[...]
  t_ref[...] = t
  o_ref[...] = acc_ref[...] + jnp.dot(
      t, w_ref[...], preferred_element_type=jnp.float32)


def _bn_body(acc_ref, b_ref, g_ref, bt_ref, o_ref):
  n = acc_ref.shape[0]
  t = acc_ref[...] + b_ref[...]
  mean = jnp.sum(t, axis=0, keepdims=True) / n
  d = t - mean
  var = jnp.sum(d * d, axis=0, keepdims=True) / n
  xn = d * jax.lax.rsqrt(var + EPS) * g_ref[...] + bt_ref[...]
  o_ref[...] = jnp.where(xn > 0, xn, ALPHA * xn)


def kernel(x, edge_idx, edge_attr, W, b, gamma, beta):
  n, din = x.shape
  kk, _, dout = W.shape
  e = edge_idx.shape[1]
  dh = din // NC

  quant = _lcm(NW * CHUNK, NS * LAP_CHUNK)
  e_pad = _cdiv(e, quant) * quant
  pad = e_pad - e
  src = jnp.concatenate([edge_idx[0], jnp.zeros((pad,), jnp.int32)])
  dst = jnp.concatenate([edge_idx[1], jnp.zeros((pad,), jnp.int32)])
  attr = jnp.concatenate([edge_attr, jnp.zeros((pad,), jnp.float32)])
  gidx = jnp.stack([NC * src, NC * src + 1])

  part = _deg_kernel(n, e_pad)(src, dst, attr)

  dis = pl.pallas_call(
      _dis_body,
      out_shape=jax.ShapeDtypeStruct((1, n), jnp.float32),
  )(part).reshape((n,))

  norm = _norm_kernel(n, e_pad)(src, dst, attr, dis)

  lap = _lap_kernel(n, e_pad, dh)
  n_chunks = e_pad // NS // LAP_CHUNK
  gidx_l = gidx.reshape(NC, NS, n_chunks, LAP_CHUNK)
  dst_l = dst.reshape(NS, n_chunks, LAP_CHUNK)
  norm_l = norm.reshape(NS, n_chunks, LAP_CHUNK)

  def lap_apply(v):
    return lap(v.reshape((NC * n, dh)), gidx_l, dst_l, norm_l).reshape((n, din))

  blk = 2000
  grid = n // blk
  row_spec = pl.BlockSpec((blk, din), lambda i: (i, 0))
  w_spec = pl.BlockSpec((din, dout), lambda i: (0, 0))

  t1 = lap_apply(x)
  out = pl.pallas_call(
      _mm_init_body,
      grid=(grid,),
      in_specs=[row_spec, row_spec, w_spec, w_spec],
      out_specs=row_spec,
      out_shape=jax.ShapeDtypeStruct((n, dout), jnp.float32),
  )(x, t1, W[0], W[1])

  tprev, tcur = x, t1
  for k in range(2, kk):
    lk = lap_apply(tcur)
    tnext, out = pl.pallas_call(
        _mm_step_body,
        grid=(grid,),
        in_specs=[row_spec, row_spec, w_spec, row_spec],
        out_specs=[row_spec, row_spec],
        out_shape=[
            jax.ShapeDtypeStruct((n, din), jnp.float32),
            jax.ShapeDtypeStruct((n, dout), jnp.float32),
        ],
    )(tprev, lk, W[k], out)
    tprev, tcur = tcur, tnext

  vec_spec = pl.BlockSpec((1, dout), lambda: (0, 0))
  full_spec = pl.BlockSpec((n, dout), lambda: (0, 0))
  return pl.pallas_call(
      _bn_body,
      in_specs=[full_spec, vec_spec, vec_spec, vec_spec],
      out_specs=full_spec,
      out_shape=jax.ShapeDtypeStruct((n, dout), jnp.float32),
  )(out, b.reshape(1, dout), gamma.reshape(1, dout), beta.reshape(1, dout))


# speedup vs baseline: 1.0510x; 1.0377x over previous
import functools

import jax
import jax.numpy as jnp
from jax import lax
from jax.experimental import pallas as pl
from jax.experimental.pallas import tpu as pltpu
from jax.experimental.pallas import tpu_sc as plsc

ALPHA = 0.01
EPS = 1e-5

NC = 2
NS = 16
NW = NC * NS
LANES = 16
CHUNK = 128
LAP_CHUNK = 80


def _cdiv(a, b):
  return (a + b - 1) // b


def _lcm(a, b):
  import math
  return a * b // math.gcd(a, b)


def _deg_kernel(n_nodes, e_pad):
  epw = e_pad // NW
  n_chunks = epw // CHUNK
  rows_per_tile = n_nodes // NS

  @functools.partial(
      pl.kernel,
      out_type=jax.ShapeDtypeStruct((NC, n_nodes, LANES), jnp.float32),
      mesh=plsc.VectorSubcoreMesh(core_axis_name="c", subcore_axis_name="s", num_cores=NC, num_subcores=NS),
      compiler_params=pltpu.CompilerParams(use_tc_tiling_on_sc=False, needs_layout_passes=False),
      scratch_types=[
          pltpu.MemorySpace.VMEM_SHARED((n_nodes, LANES), jnp.float32),
          pltpu.VMEM((rows_per_tile, LANES), jnp.float32),
          pltpu.VMEM((CHUNK, LANES), jnp.float32),
          pltpu.VMEM((CHUNK,), jnp.int32),
          pltpu.VMEM((CHUNK,), jnp.int32),
          pltpu.VMEM((CHUNK,), jnp.float32),
      ],
  )
  def deg(src_hbm, dst_hbm, attr_hbm, part_hbm,
          acc_sh, zero_v, rows_v, src_v, dst_v, attr_v):
    c = lax.axis_index("c")
    s = lax.axis_index("s")
    wid = s * NC + c
    base = wid * epw
    r0 = s * rows_per_tile

    @pl.loop(0, rows_per_tile)
    def _(r):
      zero_v[r, pl.ds(0, LANES)] = jnp.zeros((LANES,), jnp.float32)

    pltpu.sync_copy(zero_v, acc_sh.at[pl.ds(r0, rows_per_tile)])
    plsc.subcore_barrier()

    @pl.loop(0, n_chunks)
    def _(k):
      e0 = base + k * CHUNK
      pltpu.sync_copy(src_hbm.at[pl.ds(e0, CHUNK)], src_v)
      pltpu.sync_copy(dst_hbm.at[pl.ds(e0, CHUNK)], dst_v)
      pltpu.sync_copy(attr_hbm.at[pl.ds(e0, CHUNK)], attr_v)

      @pl.loop(0, CHUNK // LANES)
      def _(g):
        sl = pl.ds(g * LANES, LANES)
        ew = jnp.where(src_v[sl] == dst_v[sl], 0.0, attr_v[sl])
        for l in range(LANES):
          rows_v[g * LANES + l, pl.ds(0, LANES)] = jnp.full(
              (LANES,), ew[l], jnp.float32)

      pltpu.sync_copy(rows_v, acc_sh.at[src_v], add=True)

    plsc.subcore_barrier()
    pltpu.sync_copy(acc_sh.at[pl.ds(r0, rows_per_tile)],
                    part_hbm.at[c, pl.ds(r0, rows_per_tile)])

  return deg


def _norm_kernel(n_nodes, e_pad):
  epw = e_pad // NW
  n_chunks = epw // CHUNK

  @functools.partial(
      pl.kernel,
      out_type=jax.ShapeDtypeStruct((e_pad,), jnp.float32),
      mesh=plsc.VectorSubcoreMesh(core_axis_name="c", subcore_axis_name="s", num_cores=NC, num_subcores=NS),
      compiler_params=pltpu.CompilerParams(use_tc_tiling_on_sc=False, needs_layout_passes=False),
      scratch_types=[
          pltpu.VMEM((n_nodes,), jnp.float32),
          pltpu.VMEM((CHUNK,), jnp.int32),
          pltpu.VMEM((CHUNK,), jnp.int32),
          pltpu.VMEM((CHUNK,), jnp.float32),
          pltpu.VMEM((CHUNK,), jnp.float32),
      ],
  )
  def norm(src_hbm, dst_hbm, attr_hbm, dis_hbm, norm_hbm,
           dis_v, src_v, dst_v, attr_v, norm_v):
    c = lax.axis_index("c")
    s = lax.axis_index("s")
    wid = s * NC + c
    base = wid * epw
    pltpu.sync_copy(dis_hbm, dis_v)

    @pl.loop(0, n_chunks)
    def _(k):
      e0 = base + k * CHUNK
      pltpu.sync_copy(src_hbm.at[pl.ds(e0, CHUNK)], src_v)
      pltpu.sync_copy(dst_hbm.at[pl.ds(e0, CHUNK)], dst_v)
      pltpu.sync_copy(attr_hbm.at[pl.ds(e0, CHUNK)], attr_v)
      for j in range(CHUNK // LANES):
        sl = pl.ds(j * LANES, LANES)
        si = src_v[sl]
        di = dst_v[sl]
        av = attr_v[sl]
        dsrc = plsc.load_gather(dis_v, [si])
        ddst = plsc.load_gather(dis_v, [di])
        ew = jnp.where(si == di, 0.0, av)
        norm_v[sl] = -dsrc * ew * ddst
      pltpu.sync_copy(norm_v, norm_hbm.at[pl.ds(e0, CHUNK)])

  return norm


def _lap_kernel(n_nodes, e_pad, dh):
  ck = LAP_CHUNK
  epw = e_pad // NS
  n_chunks = epw // ck
  rows_per_tile = n_nodes // NS
  ND = 4

  @functools.partial(
      pl.kernel,
      out_type=jax.ShapeDtypeStruct((n_nodes, NC, dh), jnp.float32),
      mesh=plsc.VectorSubcoreMesh(core_axis_name="c", subcore_axis_name="s", num_cores=NC, num_subcores=NS),
      compiler_params=pltpu.CompilerParams(use_tc_tiling_on_sc=False, needs_layout_passes=False),
      scratch_types=[
          pltpu.MemorySpace.VMEM_SHARED((n_nodes, dh), jnp.float32),
          pltpu.VMEM((2, ck, dh), jnp.float32),
          pltpu.VMEM((2, ck, dh), jnp.float32),
          pltpu.VMEM((2, ck), jnp.int32),
          pltpu.VMEM((ND, ck), jnp.int32),
          pltpu.VMEM((ND, ck), jnp.float32),
          pltpu.SemaphoreType.DMA((2,)),
          pltpu.SemaphoreType.DMA((2,)),
          pltpu.SemaphoreType.DMA((2,)),
          pltpu.SemaphoreType.DMA((ND,)),
      ],
  )
  def lap(v2_hbm, gidx_hbm, dst_hbm, norm_hbm, out_hbm,
          acc_sh, rows, prod, idx_b, dst_b, norm_b, gsem, ssem, isem, ndsem):
    c = lax.axis_index("c")
    s = lax.axis_index("s")
    r0 = s * rows_per_tile

    @pl.loop(0, ck)
    def _(r):
      for j in range(dh // LANES):
        rows[0, r, pl.ds(j * LANES, LANES)] = jnp.zeros((LANES,), jnp.float32)

    nfull = rows_per_tile // ck
    rem = rows_per_tile - nfull * ck
    for z in range(nfull):
      pltpu.sync_copy(rows.at[0], acc_sh.at[pl.ds(r0 + z * ck, ck)])
    if rem:
      pltpu.sync_copy(rows.at[0, pl.ds(0, rem)],
                      acc_sh.at[pl.ds(r0 + nfull * ck, rem)])
    plsc.subcore_barrier()

    def ifetch(k, b):
      pltpu.async_copy(gidx_hbm.at[c, s, k], idx_b.at[b], isem.at[b])

    def iwait(k, b):
      pltpu.make_async_copy(gidx_hbm.at[c, s, k], idx_b.at[b],
                            isem.at[b]).wait()

    def ndfetch(k, d):
      pltpu.async_copy(dst_hbm.at[s, k], dst_b.at[d], ndsem.at[d])
      pltpu.async_copy(norm_hbm.at[s, k], norm_b.at[d], ndsem.at[d])

    def ndwait(k, d):
      pltpu.make_async_copy(dst_hbm.at[s, k], dst_b.at[d], ndsem.at[d]).wait()
      pltpu.make_async_copy(norm_hbm.at[s, k], norm_b.at[d],
                            ndsem.at[d]).wait()

    def gather(b):
      pltpu.async_copy(v2_hbm.at[idx_b.at[b]], rows.at[b], gsem.at[b])

    def gwait(b):
      pltpu.make_async_copy(v2_hbm.at[idx_b.at[b]], rows.at[b],
                            gsem.at[b]).wait()

    def scat(b, d):
      pltpu.async_copy(prod.at[b], acc_sh.at[dst_b.at[d]], ssem.at[b],
                       add=True)

    def swait(b, d):
      pltpu.make_async_copy(prod.at[b], acc_sh.at[dst_b.at[d]],
                            ssem.at[b]).wait()

    def mul(b, d):
      @plsc.parallel_loop(0, ck // LANES, unroll=2)
      def _(g):
        nv = norm_b[d, pl.ds(g * LANES, LANES)]
        for l in range(LANES):
          w = nv[l]
          r = g * LANES + l
          for j in range(dh // LANES):
            sl = pl.ds(j * LANES, LANES)
            prod[b, r, sl] = rows[b, r, sl] * w

    ifetch(0, 0)
    ndfetch(0, 0)
    ifetch(1, 1)
    ndfetch(1, 1)
    iwait(0, 0)
    gather(0)
    iwait(1, 1)
    gather(1)

    def slot(k, b, d, do_swait, do_refill):
      gwait(b)
      if do_swait:
        swait(b, (d - 2) % ND)
      if do_refill:
        ifetch(k + 2, b)
        ndfetch(k + 2, (d + 2) % ND)
      ndwait(k, d)
      mul(b, d)
      if do_refill:
        iwait(k + 2, b)
        gather(b)
      scat(b, d)

    slot(0, 0, 0, False, True)
    slot(1, 1, 1, False, True)

    @pl.loop(2, n_chunks - 2, step=ND)
    def _(kk):
      for bb in range(ND):
        slot(kk + bb, bb % 2, (2 + bb) % ND, True, True)

    slot(n_chunks - 2, 0, 2, True, False)
    slot(n_chunks - 1, 1, 3, True, False)
    swait(0, 2)
    swait(1, 3)

    plsc.subcore_barrier()
    for z in range(nfull):
      rr = r0 + z * ck
      pltpu.sync_copy(acc_sh.at[pl.ds(rr, ck)],
                      out_hbm.at[pl.ds(rr, ck), c])
    if rem:
      rr = r0 + nfull * ck
      pltpu.sync_copy(acc_sh.at[pl.ds(rr, rem)],
                      out_hbm.at[pl.ds(rr, rem), c])

  return lap


def _dis_body(part_ref, dis_ref):
  deg = part_ref[0, :, 0] + part_ref[1, :, 0]
  deg = deg[None, :]
  safe = jax.lax.rsqrt(jnp.maximum(deg, 1e-12))
  dis_ref[...] = jnp.where(deg > 0, safe, 0.0)


def _mm_init_body(x_ref, l_ref, w0_ref, w1_ref, o_ref):
  o_ref[...] = (
      jnp.dot(x_ref[...], w0_ref[...], preferred_element_type=jnp.float32)
      + jnp.dot(l_ref[...], w1_ref[...], preferred_element_type=jnp.float32))


def _mm_step_body(tprev_ref, l_ref, w_ref, acc_ref, t_ref, o_ref):
  t = 2.0 * l_ref[...] - tprev_ref[...]
  t_ref[...] = t
  o_ref[...] = acc_ref[...] + jnp.dot(
      t, w_ref[...], preferred_element_type=jnp.float32)


def _bn_body(acc_ref, b_ref, g_ref, bt_ref, o_ref):
  n = acc_ref.shape[0]
  t = acc_ref[...] + b_ref[...]
  mean = jnp.sum(t, axis=0, keepdims=True) / n
  d = t - mean
  var = jnp.sum(d * d, axis=0, keepdims=True) / n
  xn = d * jax.lax.rsqrt(var + EPS) * g_ref[...] + bt_ref[...]
  o_ref[...] = jnp.where(xn > 0, xn, ALPHA * xn)


def kernel(x, edge_idx, edge_attr, W, b, gamma, beta):
  n, din = x.shape
  kk, _, dout = W.shape
  e = edge_idx.shape[1]
  dh = din // NC

  quant = _lcm(NW * CHUNK, NS * LAP_CHUNK)
  e_pad = _cdiv(e, quant) * quant
  pad = e_pad - e
  src = jnp.concatenate([edge_idx[0], jnp.zeros((pad,), jnp.int32)])
  dst = jnp.concatenate([edge_idx[1], jnp.zeros((pad,), jnp.int32)])
  attr = jnp.concatenate([edge_attr, jnp.zeros((pad,), jnp.float32)])
  gidx = jnp.stack([NC * src, NC * src + 1])

  part = _deg_kernel(n, e_pad)(src, dst, attr)

  dis = pl.pallas_call(
      _dis_body,
      out_shape=jax.ShapeDtypeStruct((1, n), jnp.float32),
  )(part).reshape((n,))

  norm = _norm_kernel(n, e_pad)(src, dst, attr, dis)

  lap = _lap_kernel(n, e_pad, dh)
  n_chunks = e_pad // NS // LAP_CHUNK
  gidx_l = gidx.reshape(NC, NS, n_chunks, LAP_CHUNK)
  dst_l = dst.reshape(NS, n_chunks, LAP_CHUNK)
  norm_l = norm.reshape(NS, n_chunks, LAP_CHUNK)

  def lap_apply(v):
    return lap(v.reshape((NC * n, dh)), gidx_l, dst_l, norm_l).reshape((n, din))

  blk = 2000
  grid = n // blk
  row_spec = pl.BlockSpec((blk, din), lambda i: (i, 0))
  w_spec = pl.BlockSpec((din, dout), lambda i: (0, 0))

  t1 = lap_apply(x)
  out = pl.pallas_call(
      _mm_init_body,
      grid=(grid,),
      in_specs=[row_spec, row_spec, w_spec, w_spec],
      out_specs=row_spec,
      out_shape=jax.ShapeDtypeStruct((n, dout), jnp.float32),
  )(x, t1, W[0], W[1])

  tprev, tcur = x, t1
  for k in range(2, kk):
    lk = lap_apply(tcur)
    tnext, out = pl.pallas_call(
        _mm_step_body,
        grid=(grid,),
        in_specs=[row_spec, row_spec, w_spec, row_spec],
        out_specs=[row_spec, row_spec],
        out_shape=[
            jax.ShapeDtypeStruct((n, din), jnp.float32),
            jax.ShapeDtypeStruct((n, dout), jnp.float32),
        ],
    )(tprev, lk, W[k], out)
    tprev, tcur = tcur, tnext

  vec_spec = pl.BlockSpec((1, dout), lambda: (0, 0))
  full_spec = pl.BlockSpec((n, dout), lambda: (0, 0))
  return pl.pallas_call(
      _bn_body,
      in_specs=[full_spec, vec_spec, vec_spec, vec_spec],
      out_specs=full_spec,
      out_shape=jax.ShapeDtypeStruct((n, dout), jnp.float32),
  )(out, b.reshape(1, dout), gamma.reshape(1, dout), beta.reshape(1, dout))


# speedup vs baseline: 1.0515x; 1.0005x over previous
import functools

import jax
import jax.numpy as jnp
from jax import lax
from jax.experimental import pallas as pl
from jax.experimental.pallas import tpu as pltpu
from jax.experimental.pallas import tpu_sc as plsc

ALPHA = 0.01
EPS = 1e-5

NC = 2
NS = 16
NW = NC * NS
LANES = 16
CHUNK = 128
LAP_CHUNK = 128


def _cdiv(a, b):
  return (a + b - 1) // b


def _lcm(a, b):
  import math
  return a * b // math.gcd(a, b)


def _deg_kernel(n_nodes, e_pad):
  epw = e_pad // NW
  n_chunks = epw // CHUNK
  rows_per_tile = n_nodes // NS

  @functools.partial(
      pl.kernel,
      out_type=jax.ShapeDtypeStruct((NC, n_nodes, LANES), jnp.float32),
      mesh=plsc.VectorSubcoreMesh(core_axis_name="c", subcore_axis_name="s", num_cores=NC, num_subcores=NS),
      compiler_params=pltpu.CompilerParams(use_tc_tiling_on_sc=False, needs_layout_passes=False),
      scratch_types=[
          pltpu.MemorySpace.VMEM_SHARED((n_nodes, LANES), jnp.float32),
          pltpu.VMEM((rows_per_tile, LANES), jnp.float32),
          pltpu.VMEM((CHUNK, LANES), jnp.float32),
          pltpu.VMEM((CHUNK,), jnp.int32),
          pltpu.VMEM((CHUNK,), jnp.int32),
          pltpu.VMEM((CHUNK,), jnp.float32),
      ],
  )
  def deg(src_hbm, dst_hbm, attr_hbm, part_hbm,
          acc_sh, zero_v, rows_v, src_v, dst_v, attr_v):
    c = lax.axis_index("c")
    s = lax.axis_index("s")
    wid = s * NC + c
    base = wid * epw
    r0 = s * rows_per_tile

    @pl.loop(0, rows_per_tile)
    def _(r):
      zero_v[r, pl.ds(0, LANES)] = jnp.zeros((LANES,), jnp.float32)

    pltpu.sync_copy(zero_v, acc_sh.at[pl.ds(r0, rows_per_tile)])
    plsc.subcore_barrier()

    @pl.loop(0, n_chunks)
    def _(k):
      e0 = base + k * CHUNK
      pltpu.sync_copy(src_hbm.at[pl.ds(e0, CHUNK)], src_v)
      pltpu.sync_copy(dst_hbm.at[pl.ds(e0, CHUNK)], dst_v)
      pltpu.sync_copy(attr_hbm.at[pl.ds(e0, CHUNK)], attr_v)

      @pl.loop(0, CHUNK // LANES)
      def _(g):
        sl = pl.ds(g * LANES, LANES)
        ew = jnp.where(src_v[sl] == dst_v[sl], 0.0, attr_v[sl])
        for l in range(LANES):
          rows_v[g * LANES + l, pl.ds(0, LANES)] = jnp.full(
              (LANES,), ew[l], jnp.float32)

      pltpu.sync_copy(rows_v, acc_sh.at[src_v], add=True)

    plsc.subcore_barrier()
    pltpu.sync_copy(acc_sh.at[pl.ds(r0, rows_per_tile)],
                    part_hbm.at[c, pl.ds(r0, rows_per_tile)])

  return deg


def _norm_kernel(n_nodes, e_pad):
  epw = e_pad // NW
  n_chunks = epw // CHUNK

  @functools.partial(
      pl.kernel,
      out_type=jax.ShapeDtypeStruct((e_pad,), jnp.float32),
      mesh=plsc.VectorSubcoreMesh(core_axis_name="c", subcore_axis_name="s", num_cores=NC, num_subcores=NS),
      compiler_params=pltpu.CompilerParams(use_tc_tiling_on_sc=False, needs_layout_passes=False),
      scratch_types=[
          pltpu.VMEM((n_nodes,), jnp.float32),
          pltpu.VMEM((CHUNK,), jnp.int32),
          pltpu.VMEM((CHUNK,), jnp.int32),
          pltpu.VMEM((CHUNK,), jnp.float32),
          pltpu.VMEM((CHUNK,), jnp.float32),
      ],
  )
  def norm(src_hbm, dst_hbm, attr_hbm, dis_hbm, norm_hbm,
           dis_v, src_v, dst_v, attr_v, norm_v):
    c = lax.axis_index("c")
    s = lax.axis_index("s")
    wid = s * NC + c
    base = wid * epw
    pltpu.sync_copy(dis_hbm, dis_v)

    @pl.loop(0, n_chunks)
    def _(k):
      e0 = base + k * CHUNK
      pltpu.sync_copy(src_hbm.at[pl.ds(e0, CHUNK)], src_v)
      pltpu.sync_copy(dst_hbm.at[pl.ds(e0, CHUNK)], dst_v)
      pltpu.sync_copy(attr_hbm.at[pl.ds(e0, CHUNK)], attr_v)
      for j in range(CHUNK // LANES):
        sl = pl.ds(j * LANES, LANES)
        si = src_v[sl]
        di = dst_v[sl]
        av = attr_v[sl]
        dsrc = plsc.load_gather(dis_v, [si])
        ddst = plsc.load_gather(dis_v, [di])
        ew = jnp.where(si == di, 0.0, av)
        norm_v[sl] = -dsrc * ew * ddst
      pltpu.sync_copy(norm_v, norm_hbm.at[pl.ds(e0, CHUNK)])

  return norm


def _lap_kernel(n_nodes, e_pad, dh):
  ck = LAP_CHUNK
  epw = e_pad // NS
  n_chunks = epw // ck
  rows_per_tile = n_nodes // NS
  NB = 3
  ND = 4

  @functools.partial(
      pl.kernel,
      out_type=jax.ShapeDtypeStruct((n_nodes, NC, dh), jnp.float32),
      mesh=plsc.VectorSubcoreMesh(core_axis_name="c", subcore_axis_name="s", num_cores=NC, num_subcores=NS),
      compiler_params=pltpu.CompilerParams(use_tc_tiling_on_sc=False, needs_layout_passes=False),
      scratch_types=[
          pltpu.MemorySpace.VMEM_SHARED((n_nodes, dh), jnp.float32),
          pltpu.VMEM((NB, ck, dh), jnp.float32),
          pltpu.VMEM((ND, 3, ck), jnp.int32),
          pltpu.SemaphoreType.DMA((NB,)),
          pltpu.SemaphoreType.DMA((NB,)),
          pltpu.SemaphoreType.DMA((ND,)),
      ],
  )
  def lap(v2_hbm, edata_hbm, out_hbm,
          acc_sh, rows, pk, gsem, ssem, psem):
    c = lax.axis_index("c")
    s = lax.axis_index("s")
    r0 = s * rows_per_tile

    @pl.loop(0, ck)
    def _(r):
      for j in range(dh // LANES):
        rows[0, r, pl.ds(j * LANES, LANES)] = jnp.zeros((LANES,), jnp.float32)

    nfull = rows_per_tile // ck
    rem = rows_per_tile - nfull * ck
    for z in range(nfull):
      pltpu.sync_copy(rows.at[0], acc_sh.at[pl.ds(r0 + z * ck, ck)])
    if rem:
      pltpu.sync_copy(rows.at[0, pl.ds(0, rem)],
                      acc_sh.at[pl.ds(r0 + nfull * ck, rem)])
    plsc.subcore_barrier()

    def fetch(k, sl):
      pltpu.async_copy(edata_hbm.at[c, s, k], pk.at[sl], psem.at[sl])

    def fwait(k, sl):
      pltpu.make_async_copy(edata_hbm.at[c, s, k], pk.at[sl],
                            psem.at[sl]).wait()

    def gather(sl, b):
      pltpu.async_copy(v2_hbm.at[pk.at[sl, 0]], rows.at[b], gsem.at[b])

    def gwait(sl, b):
      pltpu.make_async_copy(v2_hbm.at[pk.at[sl, 0]], rows.at[b],
                            gsem.at[b]).wait()

    def scat(sl, b):
      pltpu.async_copy(rows.at[b], acc_sh.at[pk.at[sl, 1]], ssem.at[b],
                       add=True)

    def swait(sl, b):
      pltpu.make_async_copy(rows.at[b], acc_sh.at[pk.at[sl, 1]],
                            ssem.at[b]).wait()

    def mul(sl, b):
      @plsc.parallel_loop(0, ck // LANES, unroll=2)
      def _(g):
        nv = plsc.bitcast(pk[sl, 2, pl.ds(g * LANES, LANES)], jnp.float32)
        for l in range(LANES):
          w = nv[l]
          r = g * LANES + l
          for j in range(dh // LANES):
            slc = pl.ds(j * LANES, LANES)
            rows[b, r, slc] = rows[b, r, slc] * w

    def slot(k, r, do_drain, do_refill, do_fetch):
      b = r % NB
      sl = r % ND
      gwait(sl, b)
      if do_refill:
        b2 = (r + 2) % NB
        if do_drain:
          swait((r - 1) % ND, b2)
        fwait(k + 2, (r + 2) % ND)
        gather((r + 2) % ND, b2)
      if do_fetch:
        fetch(k + 3, (r + 3) % ND)
      mul(sl, b)
      scat(sl, b)

    fetch(0, 0)
    fetch(1, 1)
    fetch(2, 2)
    fwait(0, 0)
    gather(0, 0)
    fwait(1, 1)
    gather(1, 1)

    slot(0, 0, False, True, True)
    slot(1, 1, True, True, True)
    slot(2, 2, True, True, True)

    NR = 12
    main_len = ((n_chunks - 8) // NR) * NR
    @pl.loop(3, 3 + main_len, step=NR)
    def _(kk):
      for bb in range(NR):
        slot(kk + bb, (3 + bb) % NR, True, True, True)

    tail0 = 3 + main_len
    for k in range(tail0, n_chunks):
      slot(k, k % NR, True, k + 2 < n_chunks, k + 3 < n_chunks)
    for k in range(n_chunks - 3, n_chunks):
      swait(k % ND, k % NB)

    plsc.subcore_barrier()
    for z in range(nfull):
      rr = r0 + z * ck
      pltpu.sync_copy(acc_sh.at[pl.ds(rr, ck)],
                      out_hbm.at[pl.ds(rr, ck), c])
    if rem:
      rr = r0 + nfull * ck
      pltpu.sync_copy(acc_sh.at[pl.ds(rr, rem)],
                      out_hbm.at[pl.ds(rr, rem), c])

  return lap


def _dis_body(part_ref, dis_ref):
  deg = part_ref[0, :, 0] + part_ref[1, :, 0]
  deg = deg[None, :]
  safe = jax.lax.rsqrt(jnp.maximum(deg, 1e-12))
  dis_ref[...] = jnp.where(deg > 0, safe, 0.0)


def _mm_init_body(x_ref, l_ref, w0_ref, w1_ref, o_ref):
  o_ref[...] = (
      jnp.dot(x_ref[...], w0_ref[...], preferred_element_type=jnp.float32)
      + jnp.dot(l_ref[...], w1_ref[...], preferred_element_type=jnp.float32))


def _mm_step_body(tprev_ref, l_ref, w_ref, acc_ref, t_ref, o_ref):
  t = 2.0 * l_ref[...] - tprev_ref[...]
  t_ref[...] = t
  o_ref[...] = acc_ref[...] + jnp.dot(
      t, w_ref[...], preferred_element_type=jnp.float32)


def _bn_body(acc_ref, b_ref, g_ref, bt_ref, o_ref):
  n = acc_ref.shape[0]
  t = acc_ref[...] + b_ref[...]
  mean = jnp.sum(t, axis=0, keepdims=True) / n
  d = t - mean
  var = jnp.sum(d * d, axis=0, keepdims=True) / n
  xn = d * jax.lax.rsqrt(var + EPS) * g_ref[...] + bt_ref[...]
  o_ref[...] = jnp.where(xn > 0, xn, ALPHA * xn)


def kernel(x, edge_idx, edge_attr, W, b, gamma, beta):
  n, din = x.shape
  kk, _, dout = W.shape
  e = edge_idx.shape[1]
  dh = din // NC

  quant = _lcm(NW * CHUNK, NS * LAP_CHUNK)
  e_pad = _cdiv(e, quant) * quant
  pad = e_pad - e
  src = jnp.concatenate([edge_idx[0], jnp.zeros((pad,), jnp.int32)])
  dst = jnp.concatenate([edge_idx[1], jnp.zeros((pad,), jnp.int32)])
  attr = jnp.concatenate([edge_attr, jnp.zeros((pad,), jnp.float32)])
  gidx = jnp.stack([NC * src, NC * src + 1])

  part = _deg_kernel(n, e_pad)(src, dst, attr)

  dis = pl.pallas_call(
      _dis_body,
      out_shape=jax.ShapeDtypeStruct((1, n), jnp.float32),
  )(part).reshape((n,))

  norm = _norm_kernel(n, e_pad)(src, dst, attr, dis)

  lap = _lap_kernel(n, e_pad, dh)
  n_chunks = e_pad // NS // LAP_CHUNK
  gidx_l = gidx.reshape(NC, 1, NS, n_chunks, 1, LAP_CHUNK)
  dst_l = jnp.broadcast_to(
      dst.reshape(1, NS, n_chunks, 1, LAP_CHUNK),
      (NC, NS, n_chunks, 1, LAP_CHUNK))
  norm_l = jnp.broadcast_to(
      jax.lax.bitcast_convert_type(norm, jnp.int32).reshape(
          1, NS, n_chunks, 1, LAP_CHUNK),
      (NC, NS, n_chunks, 1, LAP_CHUNK))
  edata = jnp.concatenate(
      [gidx_l[:, 0], dst_l, norm_l], axis=3)

  def lap_apply(v):
    return lap(v.reshape((NC * n, dh)), edata).reshape((n, din))

  blk = 2000
  grid = n // blk
  row_spec = pl.BlockSpec((blk, din), lambda i: (i, 0))
  w_spec = pl.BlockSpec((din, dout), lambda i: (0, 0))

  t1 = lap_apply(x)
  out = pl.pallas_call(
      _mm_init_body,
      grid=(grid,),
      in_specs=[row_spec, row_spec, w_spec, w_spec],
      out_specs=row_spec,
      out_shape=jax.ShapeDtypeStruct((n, dout), jnp.float32),
  )(x, t1, W[0], W[1])

  tprev, tcur = x, t1
  for k in range(2, kk):
    lk = lap_apply(tcur)
    tnext, out = pl.pallas_call(
        _mm_step_body,
        grid=(grid,),
        in_specs=[row_spec, row_spec, w_spec, row_spec],
        out_specs=[row_spec, row_spec],
        out_shape=[
            jax.ShapeDtypeStruct((n, din), jnp.float32),
            jax.ShapeDtypeStruct((n, dout), jnp.float32),
        ],
    )(tprev, lk, W[k], out)
    tprev, tcur = tcur, tnext

  vec_spec = pl.BlockSpec((1, dout), lambda: (0, 0))
  full_spec = pl.BlockSpec((n, dout), lambda: (0, 0))
  return pl.pallas_call(
      _bn_body,
      in_specs=[full_spec, vec_spec, vec_spec, vec_spec],
      out_specs=full_spec,
      out_shape=jax.ShapeDtypeStruct((n, dout), jnp.float32),
  )(out, b.reshape(1, dout), gamma.reshape(1, dout), beta.reshape(1, dout))
